# baseline (speedup 1.0000x reference)
import functools

import jax
import jax.numpy as jnp
from jax import lax
from jax.experimental import pallas as pl
from jax.experimental.pallas import tpu as pltpu
from jax.experimental.pallas import tpu_sc as plsc

T = 4
U = 32
B = 8192
E_PER = 131072
NUM_SRC = 50000
ESIZE = 256
TU = T * U

NC, NS, L = 2, 16, 16
NW = NC * NS
NUM_NODES = 100000
W_PER_TYPE = NW // 2
EDGES_PER_TYPE_W = E_PER // W_PER_TYPE
K = 512
CH = K // 128
N_ITERS = EDGES_PER_TYPE_W // K
ACC_ROWS = B * T
ACC_PER_SUB = ACC_ROWS // NS
BASE_PER_W = B // NW


def _base_body(outnodes_hbm, ne_hbm, base_hbm, bidx_v, brows_v, sem):
    cid = lax.axis_index("c")
    sid = lax.axis_index("s")
    wid = sid * NC + cid
    for r in range(BASE_PER_W // 128):
        boff = wid * BASE_PER_W + r * 128
        pltpu.sync_copy(outnodes_hbm.at[pl.ds(boff, 128)], bidx_v)
        pltpu.async_copy(ne_hbm.at[bidx_v], brows_v, sem).wait()
        pltpu.sync_copy(brows_v, base_hbm.at[pl.ds(boff, 128)])


def _make_base_kernel():
    mesh = plsc.VectorSubcoreMesh(core_axis_name="c", subcore_axis_name="s")
    return pl.kernel(
        _base_body,
        out_type=jax.ShapeDtypeStruct((B, ESIZE), jnp.float32),
        mesh=mesh,
        compiler_params=pltpu.CompilerParams(
            needs_layout_passes=False, use_tc_tiling_on_sc=True),
        scratch_types=[
            pltpu.VMEM((128,), jnp.int32),
            pltpu.VMEM((128, ESIZE), jnp.float32),
            pltpu.SemaphoreType.DMA,
        ],
    )


def _sc_body(base_t, innodes_hbm, esrc_hbm, edst_hbm, nte_hbm,
             partials_hbm,
             esrc_v, edst_v, nvals_v,
             rowidx_0, rowidx_1, rowidx_2, rowidx_3,
             accidx_0, accidx_1, accidx_2, accidx_3,
             rows_0, rows_1, rows_2, rows_3,
             innodes_sh, acc_sh, sem):
    rowidx = (rowidx_0, rowidx_1, rowidx_2, rowidx_3)
    accidx = (accidx_0, accidx_1, accidx_2, accidx_3)
    rows = (rows_0, rows_1, rows_2, rows_3)
    cid = lax.axis_index("c")
    sid = lax.axis_index("s")
    wid = sid * NC + cid
    t_loc = wid // W_PER_TYPE
    t = base_t + t_loc
    estart = t * E_PER + (wid % W_PER_TYPE) * EDGES_PER_TYPE_W

    @pl.when(sid == 0)
    def _():
        pltpu.sync_copy(innodes_hbm, innodes_sh)

    zero16 = jnp.zeros((L,), jnp.float32)
    lanes = lax.iota(jnp.int32, L)

    def _zrow(i, carry):
        w = i * L + lanes
        plsc.store_scatter(rows_0, [w // U, w % U], zero16)
        return carry

    lax.fori_loop(0, (128 * U) // L, _zrow, 0)
    for j in range(ACC_PER_SUB // 128):
        pltpu.sync_copy(rows_0, acc_sh.at[pl.ds(sid * ACC_PER_SUB + j * 128, 128)])
    plsc.subcore_barrier()

    def _edge_iter(g, carry):
        off = estart + g * K
        pltpu.sync_copy(esrc_hbm.at[pl.ds(off, K)], esrc_v)
        pltpu.sync_copy(edst_hbm.at[pl.ds(off, K)], edst_v)
        for c in range(CH):
            pltpu.sync_copy(innodes_sh.at[esrc_v.at[pl.ds(c * 128, 128)]],
                            nvals_v.at[pl.ds(c * 128, 128)])
        for j in range(K // L):
            n = nvals_v[pl.ds(j * L, L)]
            rowidx[j // 8][pl.ds((j % 8) * L, L)] = n * 2 + t_loc
            d = edst_v[pl.ds(j * L, L)]
            accidx[j // 8][pl.ds((j % 8) * L, L)] = d * T + t
        cps = [
            pltpu.async_copy(nte_hbm.at[rowidx[c]], rows[c], sem)
            for c in range(CH)
        ]
        for cp in cps:
            cp.wait()
        for c in range(CH):
            pltpu.sync_copy(rows[c], acc_sh.at[accidx[c]], add=True)
        return carry

    lax.fori_loop(0, N_ITERS, _edge_iter, 0)
    plsc.subcore_barrier()

    pltpu.sync_copy(
        acc_sh.at[pl.ds(sid * ACC_PER_SUB, ACC_PER_SUB)],
        partials_hbm.at[cid, pl.ds(sid * ACC_PER_SUB, ACC_PER_SUB)])


def _make_sc_kernel(base_t):
    mesh = plsc.VectorSubcoreMesh(core_axis_name="c", subcore_axis_name="s")
    return pl.kernel(
        functools.partial(_sc_body, base_t),
        out_type=jax.ShapeDtypeStruct((NC, ACC_ROWS, U), jnp.float32),
        mesh=mesh,
        compiler_params=pltpu.CompilerParams(
            needs_layout_passes=False, use_tc_tiling_on_sc=False),
        scratch_types=[
            pltpu.VMEM((K,), jnp.int32),
            pltpu.VMEM((K,), jnp.int32),
            pltpu.VMEM((K,), jnp.int32),
            pltpu.VMEM((128,), jnp.int32),
            pltpu.VMEM((128,), jnp.int32),
            pltpu.VMEM((128,), jnp.int32),
            pltpu.VMEM((128,), jnp.int32),
            pltpu.VMEM((128,), jnp.int32),
            pltpu.VMEM((128,), jnp.int32),
            pltpu.VMEM((128,), jnp.int32),
            pltpu.VMEM((128,), jnp.int32),
            pltpu.VMEM((128, U), jnp.float32),
            pltpu.VMEM((128, U), jnp.float32),
            pltpu.VMEM((128, U), jnp.float32),
            pltpu.VMEM((128, U), jnp.float32),
            pltpu.VMEM_SHARED((NUM_SRC,), jnp.int32),
            pltpu.VMEM_SHARED((ACC_ROWS, U), jnp.float32),
            pltpu.SemaphoreType.DMA,
        ],
    )


BLK = 1024
NEG = -1e9


def _dense_body(p_ref, q_ref, base_ref, w1_ref, w2_ref, r_ref, wbig_ref,
                out_ref):
    nte = (p_ref[0] + p_ref[1]) + (q_ref[0] + q_ref[1])
    h = jnp.tanh(jnp.dot(nte, w1_ref[:], preferred_element_type=jnp.float32))
    mask = jnp.where(lax.broadcasted_iota(jnp.int32, (1, TU), 1) < T, 0.0, NEG)
    logits = jnp.dot(h, w2_ref[:], preferred_element_type=jnp.float32) + mask
    m = jnp.max(logits, axis=1, keepdims=True)
    e = jnp.exp(logits - m)
    att = e / jnp.sum(e, axis=1, keepdims=True)
    a128 = jnp.dot(att, r_ref[:], preferred_element_type=jnp.float32) * nte
    proj = jnp.dot(a128, wbig_ref[:], preferred_element_type=jnp.float32)
    for tt in range(T):
        v = base_ref[:] + proj[:, tt * ESIZE:(tt + 1) * ESIZE]
        ss = jnp.sum(v * v, axis=1, keepdims=True)
        inv = 1.0 / jnp.maximum(jnp.sqrt(ss), 1e-12)
        out_ref[:, tt, :] = v * inv


def _dense_call(pa, pb, base, w1bd, w2bd, rmat, wbig):
    return pl.pallas_call(
        _dense_body,
        grid=(B // BLK,),
        in_specs=[
            pl.BlockSpec((NC, BLK, TU), lambda i: (0, i, 0)),
            pl.BlockSpec((NC, BLK, TU), lambda i: (0, i, 0)),
            pl.BlockSpec((BLK, ESIZE), lambda i: (i, 0)),
            pl.BlockSpec((TU, TU), lambda i: (0, 0)),
            pl.BlockSpec((TU, TU), lambda i: (0, 0)),
            pl.BlockSpec((TU, TU), lambda i: (0, 0)),
            pl.BlockSpec((TU, T * ESIZE), lambda i: (0, 0)),
        ],
        out_specs=pl.BlockSpec((BLK, T, ESIZE), lambda i: (i, 0, 0)),
        out_shape=jax.ShapeDtypeStruct((B, T, ESIZE), jnp.float32),
    )(pa, pb, base, w1bd, w2bd, rmat, wbig)


def kernel(input_nodes, output_nodes, edge_src, edge_dst, node_embeddings,
           node_type_embeddings, trans_weights, trans_weights_s1,
           trans_weights_s2):
    input_nodes = input_nodes.astype(jnp.int32)
    output_nodes = output_nodes.astype(jnp.int32)
    edge_src = edge_src.astype(jnp.int32)
    edge_dst = edge_dst.astype(jnp.int32)

    esrc_flat = edge_src.reshape(-1)
    edst_flat = edge_dst.reshape(-1)
    nte3 = node_type_embeddings.reshape(NUM_NODES, T, U)
    tab01 = nte3[:, 0:2].reshape(2 * NUM_NODES, U)
    tab23 = nte3[:, 2:4].reshape(2 * NUM_NODES, U)
    pa = _make_sc_kernel(0)(input_nodes, esrc_flat, edst_flat, tab01)
    pb = _make_sc_kernel(2)(input_nodes, esrc_flat, edst_flat, tab23)
    base = _make_base_kernel()(output_nodes, node_embeddings)

    w1bd = jax.scipy.linalg.block_diag(*[trans_weights_s1[i] for i in range(T)])
    w2cols = jax.scipy.linalg.block_diag(*[trans_weights_s2[i] for i in range(T)])
    w2bd = jnp.pad(w2cols, ((0, 0), (0, TU - T)))
    rmat = jnp.pad(jnp.repeat(jnp.eye(T, dtype=jnp.float32), U, axis=1),
                   ((0, TU - T), (0, 0)))
    wcat = jnp.concatenate([trans_weights[i] for i in range(T)], axis=1)
    wbig = jnp.tile(wcat, (T, 1))

    return _dense_call(pa.reshape(NC, B, TU), pb.reshape(NC, B, TU), base,
                       w1bd, w2bd, rmat, wbig)


# speedup vs baseline: 1.4175x; 1.4175x over previous
import jax
import jax.numpy as jnp
from jax import lax
from jax.experimental import pallas as pl
from jax.experimental.pallas import tpu as pltpu
from jax.experimental.pallas import tpu_sc as plsc

T = 4
U = 32
B = 8192
E_PER = 131072
NUM_SRC = 50000
ESIZE = 256
TU = T * U

NC, NS, L = 2, 16, 16
NW = NC * NS
W_PER_TYPE = NW // T
EDGES_PER_TYPE_W = E_PER // W_PER_TYPE
K = 512
CH = K // 128
N_ITERS = EDGES_PER_TYPE_W // K
ACC_ROWS = B * T
ACC_PER_SUB = ACC_ROWS // NS
BASE_PER_W = B // NW


def _base_body(outnodes_hbm, ne_hbm, base_hbm, bidx_v, brows_v, sem):
    cid = lax.axis_index("c")
    sid = lax.axis_index("s")
    wid = sid * NC + cid
    for r in range(BASE_PER_W // 128):
        boff = wid * BASE_PER_W + r * 128
        pltpu.sync_copy(outnodes_hbm.at[pl.ds(boff, 128)], bidx_v)
        pltpu.async_copy(ne_hbm.at[bidx_v], brows_v, sem).wait()
        pltpu.sync_copy(brows_v, base_hbm.at[pl.ds(boff, 128)])


def _make_base_kernel():
    mesh = plsc.VectorSubcoreMesh(core_axis_name="c", subcore_axis_name="s")
    return pl.kernel(
        _base_body,
        out_type=jax.ShapeDtypeStruct((B, ESIZE), jnp.float32),
        mesh=mesh,
        compiler_params=pltpu.CompilerParams(
            needs_layout_passes=False, use_tc_tiling_on_sc=True),
        scratch_types=[
            pltpu.VMEM((128,), jnp.int32),
            pltpu.VMEM((128, ESIZE), jnp.float32),
            pltpu.SemaphoreType.DMA,
        ],
    )


def _sc_body(innodes_hbm, esrc_hbm, edst_hbm, nte_hbm,
             partials_hbm,
             esrc_v, edst_v, nvals_v,
             rowidx_0, rowidx_1, rowidx_2, rowidx_3,
             accidx_0, accidx_1, accidx_2, accidx_3,
             rows_0, rows_1, rows_2, rows_3,
             innodes_sh, acc_sh, sem):
    rowidx = (rowidx_0, rowidx_1, rowidx_2, rowidx_3)
    accidx = (accidx_0, accidx_1, accidx_2, accidx_3)
    rows = (rows_0, rows_1, rows_2, rows_3)
    cid = lax.axis_index("c")
    sid = lax.axis_index("s")
    wid = sid * NC + cid
    t = wid // W_PER_TYPE
    estart = wid * EDGES_PER_TYPE_W

    @pl.when(sid == 0)
    def _():
        pltpu.sync_copy(innodes_hbm, innodes_sh)

    zero16 = jnp.zeros((L,), jnp.float32)
    lanes = lax.iota(jnp.int32, L)

    def _zrow(i, carry):
        w = i * L + lanes
        plsc.store_scatter(rows_0, [w // U, w % U], zero16)
        return carry

    lax.fori_loop(0, (128 * U) // L, _zrow, 0)
    for j in range(ACC_PER_SUB // 128):
        pltpu.sync_copy(rows_0, acc_sh.at[pl.ds(sid * ACC_PER_SUB + j * 128, 128)])
    plsc.subcore_barrier()

    def _edge_iter(g, carry):
        off = estart + g * K
        pltpu.sync_copy(esrc_hbm.at[pl.ds(off, K)], esrc_v)
        pltpu.sync_copy(edst_hbm.at[pl.ds(off, K)], edst_v)
        for c in range(CH):
            pltpu.sync_copy(innodes_sh.at[esrc_v.at[pl.ds(c * 128, 128)]],
                            nvals_v.at[pl.ds(c * 128, 128)])
        for j in range(K // L):
            n = nvals_v[pl.ds(j * L, L)]
            rowidx[j // 8][pl.ds((j % 8) * L, L)] = n * T + t
            d = edst_v[pl.ds(j * L, L)]
            accidx[j // 8][pl.ds((j % 8) * L, L)] = d * T + t
        cps = [
            pltpu.async_copy(nte_hbm.at[rowidx[c]], rows[c], sem)
            for c in range(CH)
        ]
        for cp in cps:
            cp.wait()
        for c in range(CH):
            pltpu.sync_copy(rows[c], acc_sh.at[accidx[c]], add=True)
        return carry

    lax.fori_loop(0, N_ITERS, _edge_iter, 0)
    plsc.subcore_barrier()

    pltpu.sync_copy(
        acc_sh.at[pl.ds(sid * ACC_PER_SUB, ACC_PER_SUB)],
        partials_hbm.at[cid, pl.ds(sid * ACC_PER_SUB, ACC_PER_SUB)])


def _make_sc_kernel():
    mesh = plsc.VectorSubcoreMesh(core_axis_name="c", subcore_axis_name="s")
    return pl.kernel(
        _sc_body,
        out_type=jax.ShapeDtypeStruct((NC, ACC_ROWS, U), jnp.float32),
        mesh=mesh,
        compiler_params=pltpu.CompilerParams(
            needs_layout_passes=False, use_tc_tiling_on_sc=False),
        scratch_types=[
            pltpu.VMEM((K,), jnp.int32),
            pltpu.VMEM((K,), jnp.int32),
            pltpu.VMEM((K,), jnp.int32),
            pltpu.VMEM((128,), jnp.int32),
            pltpu.VMEM((128,), jnp.int32),
            pltpu.VMEM((128,), jnp.int32),
            pltpu.VMEM((128,), jnp.int32),
            pltpu.VMEM((128,), jnp.int32),
            pltpu.VMEM((128,), jnp.int32),
            pltpu.VMEM((128,), jnp.int32),
            pltpu.VMEM((128,), jnp.int32),
            pltpu.VMEM((128, U), jnp.float32),
            pltpu.VMEM((128, U), jnp.float32),
            pltpu.VMEM((128, U), jnp.float32),
            pltpu.VMEM((128, U), jnp.float32),
            pltpu.VMEM_SHARED((NUM_SRC,), jnp.int32),
            pltpu.VMEM_SHARED((ACC_ROWS, U), jnp.float32),
            pltpu.SemaphoreType.DMA,
        ],
    )


BLK = 1024
NEG = -1e9


def _dense_body(p_ref, base_ref, w1_ref, w2_ref, r_ref, wbig_ref, out_ref):
    nte = p_ref[0] + p_ref[1]
    h = jnp.tanh(jnp.dot(nte, w1_ref[:], preferred_element_type=jnp.float32))
    mask = jnp.where(lax.broadcasted_iota(jnp.int32, (1, TU), 1) < T, 0.0, NEG)
    logits = jnp.dot(h, w2_ref[:], preferred_element_type=jnp.float32) + mask
    m = jnp.max(logits, axis=1, keepdims=True)
    e = jnp.exp(logits - m)
    att = e / jnp.sum(e, axis=1, keepdims=True)
    a128 = jnp.dot(att, r_ref[:], preferred_element_type=jnp.float32) * nte
    proj = jnp.dot(a128, wbig_ref[:], preferred_element_type=jnp.float32)
    for tt in range(T):
        v = base_ref[:] + proj[:, tt * ESIZE:(tt + 1) * ESIZE]
        ss = jnp.sum(v * v, axis=1, keepdims=True)
        inv = 1.0 / jnp.maximum(jnp.sqrt(ss), 1e-12)
        out_ref[:, tt, :] = v * inv


def _dense_call(partials, base, w1bd, w2bd, rmat, wbig):
    return pl.pallas_call(
        _dense_body,
        grid=(B // BLK,),
        in_specs=[
            pl.BlockSpec((NC, BLK, TU), lambda i: (0, i, 0)),
            pl.BlockSpec((BLK, ESIZE), lambda i: (i, 0)),
            pl.BlockSpec((TU, TU), lambda i: (0, 0)),
            pl.BlockSpec((TU, TU), lambda i: (0, 0)),
            pl.BlockSpec((TU, TU), lambda i: (0, 0)),
            pl.BlockSpec((TU, T * ESIZE), lambda i: (0, 0)),
        ],
        out_specs=pl.BlockSpec((BLK, T, ESIZE), lambda i: (i, 0, 0)),
        out_shape=jax.ShapeDtypeStruct((B, T, ESIZE), jnp.float32),
    )(partials, base, w1bd, w2bd, rmat, wbig)


def kernel(input_nodes, output_nodes, edge_src, edge_dst, node_embeddings,
           node_type_embeddings, trans_weights, trans_weights_s1,
           trans_weights_s2):
    input_nodes = input_nodes.astype(jnp.int32)
    output_nodes = output_nodes.astype(jnp.int32)
    edge_src = edge_src.astype(jnp.int32)
    edge_dst = edge_dst.astype(jnp.int32)

    partials = _make_sc_kernel()(
        input_nodes, edge_src.reshape(-1), edge_dst.reshape(-1),
        node_type_embeddings)
    base = _make_base_kernel()(output_nodes, node_embeddings)

    w1bd = jax.scipy.linalg.block_diag(*[trans_weights_s1[i] for i in range(T)])
    w2cols = jax.scipy.linalg.block_diag(*[trans_weights_s2[i] for i in range(T)])
    w2bd = jnp.pad(w2cols, ((0, 0), (0, TU - T)))
    rmat = jnp.pad(jnp.repeat(jnp.eye(T, dtype=jnp.float32), U, axis=1),
                   ((0, TU - T), (0, 0)))
    wcat = jnp.concatenate([trans_weights[i] for i in range(T)], axis=1)
    wbig = jnp.tile(wcat, (T, 1))

    return _dense_call(partials.reshape(NC, B, TU), base,
                       w1bd, w2bd, rmat, wbig)
